# baseline (speedup 1.0000x reference)
import functools

import jax
import jax.numpy as jnp
from jax import lax
from jax.experimental import pallas as pl
from jax.experimental.pallas import tpu as pltpu
from jax.experimental.pallas import tpu_sc as plsc


def _make_sc_gather(total, vocab, dim, chunk):
    info = plsc.get_sparse_core_info()
    nc, ns = info.num_cores, info.num_subcores
    nw = nc * ns
    assert total % (nw * chunk) == 0
    per_w = total // nw
    n_iter = per_w // chunk
    mesh = plsc.VectorSubcoreMesh(core_axis_name="c", subcore_axis_name="s")

    @functools.partial(
        pl.kernel,
        mesh=mesh,
        out_type=jax.ShapeDtypeStruct((total, dim), jnp.float32),
        compiler_params=pltpu.CompilerParams(use_tc_tiling_on_sc=False),
        scratch_types=[
            pltpu.VMEM((chunk,), jnp.int32),
            pltpu.VMEM((chunk, dim), jnp.float32),
            pltpu.SemaphoreType.DMA,
        ],
    )
    def gather(idx_hbm, table_hbm, emb_hbm, idx_v, rows_v, sem):
        wid = lax.axis_index("s") * nc + lax.axis_index("c")
        w_base = wid * per_w

        def body(i, carry):
            base = w_base + i * chunk
            pltpu.sync_copy(idx_hbm.at[pl.ds(base, chunk)], idx_v)
            pltpu.async_copy(table_hbm.at[idx_v], rows_v, sem).wait()
            pltpu.sync_copy(rows_v, emb_hbm.at[pl.ds(base, chunk)])
            return carry

        lax.fori_loop(0, n_iter, body, 0)

    return gather


def _mm_body(emb_ref, w_ref, b_ref, out_ref):
    out_ref[...] = (
        jnp.dot(emb_ref[...], w_ref[...], preferred_element_type=jnp.float32)
        + b_ref[...]
    )


def _project(emb, wt, b2, block_m):
    total, dim = emb.shape
    out_dim = wt.shape[1]
    grid = (total // block_m,)
    return pl.pallas_call(
        _mm_body,
        grid=grid,
        in_specs=[
            pl.BlockSpec((block_m, dim), lambda i: (i, 0)),
            pl.BlockSpec((dim, out_dim), lambda i: (0, 0)),
            pl.BlockSpec((1, out_dim), lambda i: (0, 0)),
        ],
        out_specs=pl.BlockSpec((block_m, out_dim), lambda i: (i, 0)),
        out_shape=jax.ShapeDtypeStruct((total, out_dim), jnp.float32),
    )(emb, wt, b2)


def kernel(x, table, W, b):
    batch, hist = x.shape
    vocab, dim = table.shape
    out_dim = W.shape[0]
    total = batch * hist

    idx = x.reshape(-1).astype(jnp.int32)
    scale = jnp.sqrt(jnp.float32(dim))
    wt = (W * scale).T
    b2 = b.reshape(1, out_dim)

    gather = _make_sc_gather(total, vocab, dim, chunk=512)
    emb = gather(idx, table)

    out = _project(emb, wt, b2, block_m=4096)
    return out.reshape(batch, hist, out_dim)
